# baseline (speedup 1.0000x reference)
import jax
import jax.numpy as jnp
from jax import lax
from jax.experimental import pallas as pl
from jax.experimental.pallas import tpu as pltpu
from jax.experimental.pallas import tpu_sc as plsc

NC = 2
NS = 16
L = 16

N = 32768
DM = 768
HALF = DM // 2
R = 4096
J = 28672

DEFAULT_SLOT = R
TOK_PER_SUB = N // NS
R_PER_TILE = R // (NC * NS)
J_PER_TILE = J // (NC * NS)
CHA = 64
J_PER_SUB = J // NS
CH = 128
FIX_CAP = J_PER_SUB + CH
OUT_PAD_ROW = 2 * J


def _iota16():
  return lax.iota(jnp.int32, L)


def _gather_pos_body(x_ref, idr_ref, ids_ref,
                     red_ref, xt1_ref, cntr_ref, slots_ref,
                     tbuf, posslice, cntslice, pos_local, cnt_local,
                     idxbuf, rowbuf, lkpbuf, cntf,
                     shared_pos, shared_cnt, sem):
  c = lax.axis_index("c")
  s = lax.axis_index("s")
  wid = s * NC + c

  def gchunk(q, _):
    base = wid * R_PER_TILE + q * CHA
    pltpu.sync_copy(idr_ref.at[pl.ds(base, CHA)], idxbuf)
    pltpu.async_copy(x_ref.at[idxbuf], rowbuf, sem).wait()
    pltpu.sync_copy(rowbuf, red_ref.at[pl.ds(base, CHA)])

    def bump(k, _):
      idxbuf[pl.ds(k * L, L)] = idxbuf[pl.ds(k * L, L)] + 1
      return 0
    lax.fori_loop(0, CHA // L, bump, 0)
    pltpu.async_copy(x_ref.at[idxbuf], rowbuf, sem).wait()
    pltpu.sync_copy(rowbuf, xt1_ref.at[pl.ds(base, CHA)])
    return 0
  lax.fori_loop(0, R_PER_TILE // CHA, gchunk, 0)

  pltpu.sync_copy(idr_ref, tbuf)
  lo = s * TOK_PER_SUB

  def init_body(k, _):
    posslice[pl.ds(k * L, L)] = jnp.full((L,), DEFAULT_SLOT, jnp.int32)
    cntslice[pl.ds(k * L, L)] = jnp.zeros((L,), jnp.int32)
    return 0
  lax.fori_loop(0, TOK_PER_SUB // L, init_body, 0)

  def scat_body(k, _):
    tv = tbuf[pl.ds(k * L, L)] + 1
    sl = _iota16() + k * L
    m = (tv >= lo) & (tv < lo + TOK_PER_SUB)
    idx = jnp.where(m, tv - lo, 0)
    plsc.store_scatter(posslice, [idx], sl, mask=m)
    plsc.addupdate_scatter(cntslice, [idx], jnp.ones((L,), jnp.int32), mask=m)
    return 0
  lax.fori_loop(0, R // L, scat_body, 0)

  pltpu.sync_copy(posslice, shared_pos.at[pl.ds(lo, TOK_PER_SUB)])
  pltpu.sync_copy(cntslice, shared_cnt.at[pl.ds(lo, TOK_PER_SUB)])
  plsc.subcore_barrier()
  pltpu.sync_copy(shared_pos, pos_local)
  pltpu.sync_copy(shared_cnt, cnt_local)

  def lkr(k, _):
    tv = tbuf[pl.ds(wid * R_PER_TILE + k * L, L)] + 1
    cv = plsc.load_gather(cnt_local, [tv])
    cntf[pl.ds(k * L, L)] = cv.astype(jnp.float32)
    return 0
  lax.fori_loop(0, R_PER_TILE // L, lkr, 0)
  pltpu.sync_copy(cntf, cntr_ref.at[pl.ds(wid * R_PER_TILE, R_PER_TILE)])

  pltpu.sync_copy(ids_ref.at[pl.ds(wid * J_PER_TILE, J_PER_TILE)], lkpbuf)

  def lks(k, _):
    sv = lkpbuf[pl.ds(k * L, L)]
    lkpbuf[pl.ds(k * L, L)] = plsc.load_gather(pos_local, [sv])
    return 0
  lax.fori_loop(0, J_PER_TILE // L, lks, 0)
  pltpu.sync_copy(lkpbuf, slots_ref.at[pl.ds(wid * J_PER_TILE, J_PER_TILE)])


def _sc_gather_pos(x_flat, ids_to_reduce, ids_to_save):
  mesh = plsc.VectorSubcoreMesh(core_axis_name="c", subcore_axis_name="s")
  return pl.kernel(
      _gather_pos_body,
      out_type=[
          jax.ShapeDtypeStruct((R, DM), jnp.float32),
          jax.ShapeDtypeStruct((R, DM), jnp.float32),
          jax.ShapeDtypeStruct((R,), jnp.float32),
          jax.ShapeDtypeStruct((J,), jnp.int32),
      ],
      mesh=mesh,
      compiler_params=pltpu.CompilerParams(needs_layout_passes=False),
      scratch_types=[
          pltpu.VMEM((R,), jnp.int32),
          pltpu.VMEM((TOK_PER_SUB,), jnp.int32),
          pltpu.VMEM((TOK_PER_SUB,), jnp.int32),
          pltpu.VMEM((N,), jnp.int32),
          pltpu.VMEM((N,), jnp.int32),
          pltpu.VMEM((CHA,), jnp.int32),
          pltpu.VMEM((CHA, DM), jnp.float32),
          pltpu.VMEM((J_PER_TILE,), jnp.int32),
          pltpu.VMEM((R_PER_TILE,), jnp.float32),
          pltpu.VMEM_SHARED((N,), jnp.int32),
          pltpu.VMEM_SHARED((N,), jnp.int32),
          pltpu.SemaphoreType.DMA,
      ],
  )(x_flat, ids_to_reduce, ids_to_save)


def _mm_body(a_ref, w_ref, xt1_ref, cnt_ref, o_ref):
  prod = lax.dot_general(
      a_ref[...], w_ref[...], (((1,), (1,)), ((), ())),
      preferred_element_type=jnp.float32)
  o_ref[...] = prod * cnt_ref[0, 0, :][:, None] + xt1_ref[...]


def _tc_matmul(reduced, w, xt1, cnt_r):
  return pl.pallas_call(
      _mm_body,
      grid=(16,),
      in_specs=[
          pl.BlockSpec((R // 16, DM), lambda i: (i, 0)),
          pl.BlockSpec((DM, DM), lambda i: (0, 0)),
          pl.BlockSpec((R // 16, DM), lambda i: (i, 0)),
          pl.BlockSpec((1, 1, R // 16), lambda i: (i, 0, 0)),
      ],
      out_specs=pl.BlockSpec((R // 16, DM), lambda i: (i, 0)),
      out_shape=jax.ShapeDtypeStruct((R, DM), jnp.float32),
  )(reduced, w, xt1, cnt_r.reshape(16, 1, R // 16))


def _merge_body(x2_ref, nv2_ref, ids_ref, slots_ref, out_ref,
                rowbuf, idxbuf, schunk, sschunk, tmpidx, tmpslot,
                fixslot, fixoidx, sem):
  c = lax.axis_index("c")
  s = lax.axis_index("s")

  def prefill(k, _):
    fixslot[pl.ds(k * L, L)] = jnp.full((L,), c, jnp.int32)
    fixoidx[pl.ds(k * L, L)] = jnp.full((L,), OUT_PAD_ROW, jnp.int32)
    return 0
  lax.fori_loop(0, FIX_CAP // L, prefill, 0)

  jbase = s * J_PER_SUB

  def p3a(q, cnt):
    base = jbase + q * CH
    pltpu.sync_copy(ids_ref.at[pl.ds(base, CH)], schunk)
    pltpu.sync_copy(slots_ref.at[pl.ds(base, CH)], sschunk)

    def mkidx(k, cnt):
      sv = schunk[pl.ds(k * L, L)]
      idxbuf[pl.ds(k * L, L)] = 2 * sv + c
      pv = sschunk[pl.ds(k * L, L)]
      m = pv != DEFAULT_SLOT
      jt = base + k * L + _iota16()
      plsc.store_compressed(fixslot.at[pl.ds(cnt, L)], 2 * pv + c, mask=m)
      plsc.store_compressed(fixoidx.at[pl.ds(cnt, L)], 2 * jt + c, mask=m)
      return cnt + jnp.sum(m.astype(jnp.int32))
    cnt = lax.fori_loop(0, CH // L, mkidx, cnt)

    pltpu.async_copy(x2_ref.at[idxbuf], rowbuf, sem).wait()

    def mko(k, _):
      jt = base + k * L + _iota16()
      tmpidx[pl.ds(k * L, L)] = 2 * jt + c
      return 0
    lax.fori_loop(0, CH // L, mko, 0)
    pltpu.async_copy(rowbuf, out_ref.at[tmpidx], sem).wait()
    return cnt
  cnt = lax.fori_loop(0, J_PER_SUB // CH, p3a, 0)

  trips = lax.div(cnt + (CH - 1), CH)

  def fixb(q, _):
    def cp(k, _):
      tmpslot[pl.ds(k * L, L)] = fixslot[pl.ds(q * CH + k * L, L)]
      tmpidx[pl.ds(k * L, L)] = fixoidx[pl.ds(q * CH + k * L, L)]
      return 0
    lax.fori_loop(0, CH // L, cp, 0)
    pltpu.async_copy(nv2_ref.at[tmpslot], rowbuf, sem).wait()
    pltpu.async_copy(rowbuf, out_ref.at[tmpidx], sem).wait()
    return 0
  lax.fori_loop(0, trips, fixb, 0)


def _sc_merge(x2, nv2, ids_to_save, slot_s):
  mesh = plsc.VectorSubcoreMesh(core_axis_name="c", subcore_axis_name="s")
  return pl.kernel(
      _merge_body,
      out_type=jax.ShapeDtypeStruct((2 * J + 16, HALF), jnp.float32),
      mesh=mesh,
      compiler_params=pltpu.CompilerParams(needs_layout_passes=False),
      scratch_types=[
          pltpu.VMEM((CH, HALF), jnp.float32),
          pltpu.VMEM((CH,), jnp.int32),
          pltpu.VMEM((CH,), jnp.int32),
          pltpu.VMEM((CH,), jnp.int32),
          pltpu.VMEM((CH,), jnp.int32),
          pltpu.VMEM((CH,), jnp.int32),
          pltpu.VMEM((FIX_CAP,), jnp.int32),
          pltpu.VMEM((FIX_CAP,), jnp.int32),
          pltpu.SemaphoreType.DMA,
      ],
  )(x2, nv2, ids_to_save, slot_s)


@jax.jit
def kernel(x, ids_to_save, ids_to_reduce, W):
  B, S, dm = x.shape
  x_flat = x.reshape(-1, dm)
  reduced, xt1, cnt_r, slot_s = _sc_gather_pos(
      x_flat, ids_to_reduce, ids_to_save)
  newvals = _tc_matmul(reduced, W, xt1, cnt_r)
  x2 = x_flat.reshape(-1, HALF)
  nv2 = newvals.reshape(-1, HALF)
  out2 = _sc_merge(x2, nv2, ids_to_save, slot_s)
  return out2[:2 * J].reshape(B, -1, dm)


# speedup vs baseline: 3.1236x; 3.1236x over previous
import jax
import jax.numpy as jnp
from jax import lax
from jax.experimental import pallas as pl
from jax.experimental.pallas import tpu as pltpu
from jax.experimental.pallas import tpu_sc as plsc

NC = 2
NS = 16
L = 16

N = 32768
DM = 768
HALF = DM // 2
R = 4096
J = 28672

DEFAULT_SLOT = R
TOK_PER_SUB = N // NS
R_PER_TILE = R // (NC * NS)
J_PER_TILE = J // (NC * NS)
CHA = 64
J_PER_W = J // (NC * NS)
CH = 128
FIX_CAP = J_PER_W + CH


def _iota16():
  return lax.iota(jnp.int32, L)


def _gather_pos_body(x_ref, idr_ref, ids_ref,
                     red_ref, xt1_ref, cntr_ref, slots_ref,
                     tbuf, posslice, cntslice, pos_local, cnt_local,
                     idxbuf, rowbuf, lkpbuf, cntf,
                     shared_pos, shared_cnt, sem):
  c = lax.axis_index("c")
  s = lax.axis_index("s")
  wid = s * NC + c

  def gchunk(q, _):
    base = wid * R_PER_TILE + q * CHA
    pltpu.sync_copy(idr_ref.at[pl.ds(base, CHA)], idxbuf)
    pltpu.async_copy(x_ref.at[idxbuf], rowbuf, sem).wait()
    pltpu.sync_copy(rowbuf, red_ref.at[pl.ds(base, CHA)])

    def bump(k, _):
      idxbuf[pl.ds(k * L, L)] = idxbuf[pl.ds(k * L, L)] + 1
      return 0
    lax.fori_loop(0, CHA // L, bump, 0)
    pltpu.async_copy(x_ref.at[idxbuf], rowbuf, sem).wait()
    pltpu.sync_copy(rowbuf, xt1_ref.at[pl.ds(base, CHA)])
    return 0
  lax.fori_loop(0, R_PER_TILE // CHA, gchunk, 0)

  pltpu.sync_copy(idr_ref, tbuf)
  lo = s * TOK_PER_SUB

  def init_body(k, _):
    posslice[pl.ds(k * L, L)] = jnp.full((L,), DEFAULT_SLOT, jnp.int32)
    cntslice[pl.ds(k * L, L)] = jnp.zeros((L,), jnp.int32)
    return 0
  lax.fori_loop(0, TOK_PER_SUB // L, init_body, 0)

  def scat_body(k, _):
    tv = tbuf[pl.ds(k * L, L)] + 1
    sl = _iota16() + k * L
    m = (tv >= lo) & (tv < lo + TOK_PER_SUB)
    idx = jnp.where(m, tv - lo, 0)
    plsc.store_scatter(posslice, [idx], sl, mask=m)
    plsc.addupdate_scatter(cntslice, [idx], jnp.ones((L,), jnp.int32), mask=m)
    return 0
  lax.fori_loop(0, R // L, scat_body, 0)

  pltpu.sync_copy(posslice, shared_pos.at[pl.ds(lo, TOK_PER_SUB)])
  pltpu.sync_copy(cntslice, shared_cnt.at[pl.ds(lo, TOK_PER_SUB)])
  plsc.subcore_barrier()
  pltpu.sync_copy(shared_pos, pos_local)
  pltpu.sync_copy(shared_cnt, cnt_local)

  def lkr(k, _):
    tv = tbuf[pl.ds(wid * R_PER_TILE + k * L, L)] + 1
    cv = plsc.load_gather(cnt_local, [tv])
    cntf[pl.ds(k * L, L)] = cv.astype(jnp.float32)
    return 0
  lax.fori_loop(0, R_PER_TILE // L, lkr, 0)
  pltpu.sync_copy(cntf, cntr_ref.at[pl.ds(wid * R_PER_TILE, R_PER_TILE)])

  pltpu.sync_copy(ids_ref.at[pl.ds(wid * J_PER_TILE, J_PER_TILE)], lkpbuf)

  def lks(k, _):
    sv = lkpbuf[pl.ds(k * L, L)]
    lkpbuf[pl.ds(k * L, L)] = plsc.load_gather(pos_local, [sv])
    return 0
  lax.fori_loop(0, J_PER_TILE // L, lks, 0)
  pltpu.sync_copy(lkpbuf, slots_ref.at[pl.ds(wid * J_PER_TILE, J_PER_TILE)])


def _sc_gather_pos(x_flat, ids_to_reduce, ids_to_save):
  mesh = plsc.VectorSubcoreMesh(core_axis_name="c", subcore_axis_name="s")
  return pl.kernel(
      _gather_pos_body,
      out_type=[
          jax.ShapeDtypeStruct((R, DM), jnp.float32),
          jax.ShapeDtypeStruct((R, DM), jnp.float32),
          jax.ShapeDtypeStruct((R,), jnp.float32),
          jax.ShapeDtypeStruct((J,), jnp.int32),
      ],
      mesh=mesh,
      compiler_params=pltpu.CompilerParams(needs_layout_passes=False),
      scratch_types=[
          pltpu.VMEM((R,), jnp.int32),
          pltpu.VMEM((TOK_PER_SUB,), jnp.int32),
          pltpu.VMEM((TOK_PER_SUB,), jnp.int32),
          pltpu.VMEM((N,), jnp.int32),
          pltpu.VMEM((N,), jnp.int32),
          pltpu.VMEM((CHA,), jnp.int32),
          pltpu.VMEM((CHA, DM), jnp.float32),
          pltpu.VMEM((J_PER_TILE,), jnp.int32),
          pltpu.VMEM((R_PER_TILE,), jnp.float32),
          pltpu.VMEM_SHARED((N,), jnp.int32),
          pltpu.VMEM_SHARED((N,), jnp.int32),
          pltpu.SemaphoreType.DMA,
      ],
  )(x_flat, ids_to_reduce, ids_to_save)


def _mm_body(a_ref, w_ref, xt1_ref, cnt_ref, o_ref):
  prod = lax.dot_general(
      a_ref[...], w_ref[...], (((1,), (1,)), ((), ())),
      preferred_element_type=jnp.float32)
  o_ref[...] = prod * cnt_ref[0, 0, :][:, None] + xt1_ref[...]


def _tc_matmul(reduced, w, xt1, cnt_r):
  return pl.pallas_call(
      _mm_body,
      grid=(16,),
      in_specs=[
          pl.BlockSpec((R // 16, DM), lambda i: (i, 0)),
          pl.BlockSpec((DM, DM), lambda i: (0, 0)),
          pl.BlockSpec((R // 16, DM), lambda i: (i, 0)),
          pl.BlockSpec((1, 1, R // 16), lambda i: (i, 0, 0)),
      ],
      out_specs=pl.BlockSpec((R // 16, DM), lambda i: (i, 0)),
      out_shape=jax.ShapeDtypeStruct((R, DM), jnp.float32),
  )(reduced, w, xt1, cnt_r.reshape(16, 1, R // 16))


def _merge_body(x_ref, nv_ref, ids_ref, slots_ref, out_ref,
                rowbuf, idxbuf, sschunk, tmpidx, tmpslot,
                fixslot, fixoidx, sem):
  c = lax.axis_index("c")
  s = lax.axis_index("s")
  wid = s * NC + c
  jbase = wid * J_PER_W

  def p3a(q, cnt):
    base = jbase + q * CH
    pltpu.sync_copy(ids_ref.at[pl.ds(base, CH)], idxbuf)
    pltpu.sync_copy(slots_ref.at[pl.ds(base, CH)], sschunk)

    def mkidx(k, cnt):
      pv = sschunk[pl.ds(k * L, L)]
      m = pv != DEFAULT_SLOT
      jt = base + k * L + _iota16()
      plsc.store_compressed(fixslot.at[pl.ds(cnt, L)], pv, mask=m)
      plsc.store_compressed(fixoidx.at[pl.ds(cnt, L)], jt, mask=m)
      return cnt + jnp.sum(m.astype(jnp.int32))
    cnt = lax.fori_loop(0, CH // L, mkidx, cnt)

    pltpu.async_copy(x_ref.at[idxbuf], rowbuf, sem).wait()
    pltpu.sync_copy(rowbuf, out_ref.at[pl.ds(base, CH)])
    return cnt
  cnt = lax.fori_loop(0, J_PER_W // CH, p3a, 0)

  @pl.when(cnt > 0)
  def _():
    z = jnp.zeros((L,), jnp.int32)
    b_slot = plsc.load_gather(fixslot, [z])
    b_oidx = plsc.load_gather(fixoidx, [z])

    def pf(k, _):
      fixslot[pl.ds(cnt + k * L, L)] = b_slot
      fixoidx[pl.ds(cnt + k * L, L)] = b_oidx
      return 0
    lax.fori_loop(0, CH // L, pf, 0)

  trips = lax.div(cnt + (CH - 1), CH)

  def fixb(q, _):
    def cp(k, _):
      tmpslot[pl.ds(k * L, L)] = fixslot[pl.ds(q * CH + k * L, L)]
      tmpidx[pl.ds(k * L, L)] = fixoidx[pl.ds(q * CH + k * L, L)]
      return 0
    lax.fori_loop(0, CH // L, cp, 0)
    pltpu.async_copy(nv_ref.at[tmpslot], rowbuf, sem).wait()
    pltpu.async_copy(rowbuf, out_ref.at[tmpidx], sem).wait()
    return 0
  lax.fori_loop(0, trips, fixb, 0)


def _sc_merge(x_flat, newvals, ids_to_save, slot_s):
  mesh = plsc.VectorSubcoreMesh(core_axis_name="c", subcore_axis_name="s")
  return pl.kernel(
      _merge_body,
      out_type=jax.ShapeDtypeStruct((J, DM), jnp.float32),
      mesh=mesh,
      compiler_params=pltpu.CompilerParams(needs_layout_passes=False),
      scratch_types=[
          pltpu.VMEM((CH, DM), jnp.float32),
          pltpu.VMEM((CH,), jnp.int32),
          pltpu.VMEM((CH,), jnp.int32),
          pltpu.VMEM((CH,), jnp.int32),
          pltpu.VMEM((CH,), jnp.int32),
          pltpu.VMEM((FIX_CAP,), jnp.int32),
          pltpu.VMEM((FIX_CAP,), jnp.int32),
          pltpu.SemaphoreType.DMA,
      ],
  )(x_flat, newvals, ids_to_save, slot_s)


@jax.jit
def kernel(x, ids_to_save, ids_to_reduce, W):
  B, S, dm = x.shape
  x_flat = x.reshape(-1, dm)
  reduced, xt1, cnt_r, slot_s = _sc_gather_pos(
      x_flat, ids_to_reduce, ids_to_save)
  newvals = _tc_matmul(reduced, W, xt1, cnt_r)
  out = _sc_merge(x_flat, newvals, ids_to_save, slot_s)
  return out.reshape(B, -1, dm)


# speedup vs baseline: 3.2427x; 1.0381x over previous
import jax
import jax.numpy as jnp
from jax import lax
from jax.experimental import pallas as pl
from jax.experimental.pallas import tpu as pltpu
from jax.experimental.pallas import tpu_sc as plsc

NC = 2
NS = 16
L = 16

N = 32768
DM = 768
HALF = DM // 2
R = 4096
J = 28672

DEFAULT_SLOT = R
TOK_PER_SUB = N // NS
R_PER_TILE = R // (NC * NS)
J_PER_TILE = J // (NC * NS)
CHA = 64
J_PER_W = J // (NC * NS)
CH = 64
FIX_CAP = J_PER_W + CH


def _iota16():
  return lax.iota(jnp.int32, L)


def _gather_pos_body(x_ref, idr_ref, ids_ref,
                     red_ref, xt1_ref, cntr_ref, slots_ref,
                     tbuf, posslice, cntslice, pos_local, cnt_local,
                     idxbuf, rowbuf, lkpbuf, cntf,
                     shared_pos, shared_cnt, sem):
  c = lax.axis_index("c")
  s = lax.axis_index("s")
  wid = s * NC + c

  def gchunk(q, _):
    base = wid * R_PER_TILE + q * CHA
    pltpu.sync_copy(idr_ref.at[pl.ds(base, CHA)], idxbuf)
    pltpu.async_copy(x_ref.at[idxbuf], rowbuf, sem).wait()
    pltpu.sync_copy(rowbuf, red_ref.at[pl.ds(base, CHA)])

    def bump(k, _):
      idxbuf[pl.ds(k * L, L)] = idxbuf[pl.ds(k * L, L)] + 1
      return 0
    lax.fori_loop(0, CHA // L, bump, 0)
    pltpu.async_copy(x_ref.at[idxbuf], rowbuf, sem).wait()
    pltpu.sync_copy(rowbuf, xt1_ref.at[pl.ds(base, CHA)])
    return 0
  lax.fori_loop(0, R_PER_TILE // CHA, gchunk, 0)

  pltpu.sync_copy(idr_ref, tbuf)
  lo = s * TOK_PER_SUB

  def init_body(k, _):
    posslice[pl.ds(k * L, L)] = jnp.full((L,), DEFAULT_SLOT, jnp.int32)
    cntslice[pl.ds(k * L, L)] = jnp.zeros((L,), jnp.int32)
    return 0
  lax.fori_loop(0, TOK_PER_SUB // L, init_body, 0)

  def scat_body(k, _):
    tv = tbuf[pl.ds(k * L, L)] + 1
    sl = _iota16() + k * L
    m = (tv >= lo) & (tv < lo + TOK_PER_SUB)
    idx = jnp.where(m, tv - lo, 0)
    plsc.store_scatter(posslice, [idx], sl, mask=m)
    plsc.addupdate_scatter(cntslice, [idx], jnp.ones((L,), jnp.int32), mask=m)
    return 0
  lax.fori_loop(0, R // L, scat_body, 0)

  pltpu.sync_copy(posslice, shared_pos.at[pl.ds(lo, TOK_PER_SUB)])
  pltpu.sync_copy(cntslice, shared_cnt.at[pl.ds(lo, TOK_PER_SUB)])
  plsc.subcore_barrier()
  pltpu.sync_copy(shared_pos, pos_local)
  pltpu.sync_copy(shared_cnt, cnt_local)

  def lkr(k, _):
    tv = tbuf[pl.ds(wid * R_PER_TILE + k * L, L)] + 1
    cv = plsc.load_gather(cnt_local, [tv])
    cntf[pl.ds(k * L, L)] = cv.astype(jnp.float32)
    return 0
  lax.fori_loop(0, R_PER_TILE // L, lkr, 0)
  pltpu.sync_copy(cntf, cntr_ref.at[pl.ds(wid * R_PER_TILE, R_PER_TILE)])

  pltpu.sync_copy(ids_ref.at[pl.ds(wid * J_PER_TILE, J_PER_TILE)], lkpbuf)

  def lks(k, _):
    sv = lkpbuf[pl.ds(k * L, L)]
    lkpbuf[pl.ds(k * L, L)] = plsc.load_gather(pos_local, [sv])
    return 0
  lax.fori_loop(0, J_PER_TILE // L, lks, 0)
  pltpu.sync_copy(lkpbuf, slots_ref.at[pl.ds(wid * J_PER_TILE, J_PER_TILE)])


def _sc_gather_pos(x_flat, ids_to_reduce, ids_to_save):
  mesh = plsc.VectorSubcoreMesh(core_axis_name="c", subcore_axis_name="s")
  return pl.kernel(
      _gather_pos_body,
      out_type=[
          jax.ShapeDtypeStruct((R, DM), jnp.float32),
          jax.ShapeDtypeStruct((R, DM), jnp.float32),
          jax.ShapeDtypeStruct((R,), jnp.float32),
          jax.ShapeDtypeStruct((J,), jnp.int32),
      ],
      mesh=mesh,
      compiler_params=pltpu.CompilerParams(needs_layout_passes=False),
      scratch_types=[
          pltpu.VMEM((R,), jnp.int32),
          pltpu.VMEM((TOK_PER_SUB,), jnp.int32),
          pltpu.VMEM((TOK_PER_SUB,), jnp.int32),
          pltpu.VMEM((N,), jnp.int32),
          pltpu.VMEM((N,), jnp.int32),
          pltpu.VMEM((CHA,), jnp.int32),
          pltpu.VMEM((CHA, DM), jnp.float32),
          pltpu.VMEM((J_PER_TILE,), jnp.int32),
          pltpu.VMEM((R_PER_TILE,), jnp.float32),
          pltpu.VMEM_SHARED((N,), jnp.int32),
          pltpu.VMEM_SHARED((N,), jnp.int32),
          pltpu.SemaphoreType.DMA,
      ],
  )(x_flat, ids_to_reduce, ids_to_save)


def _mm_body(a_ref, w_ref, xt1_ref, cnt_ref, o_ref):
  prod = lax.dot_general(
      a_ref[...], w_ref[...], (((1,), (1,)), ((), ())),
      preferred_element_type=jnp.float32)
  o_ref[...] = prod * cnt_ref[0, 0, :][:, None] + xt1_ref[...]


def _tc_matmul(reduced, w, xt1, cnt_r):
  return pl.pallas_call(
      _mm_body,
      grid=(16,),
      in_specs=[
          pl.BlockSpec((R // 16, DM), lambda i: (i, 0)),
          pl.BlockSpec((DM, DM), lambda i: (0, 0)),
          pl.BlockSpec((R // 16, DM), lambda i: (i, 0)),
          pl.BlockSpec((1, 1, R // 16), lambda i: (i, 0, 0)),
      ],
      out_specs=pl.BlockSpec((R // 16, DM), lambda i: (i, 0)),
      out_shape=jax.ShapeDtypeStruct((R, DM), jnp.float32),
  )(reduced, w, xt1, cnt_r.reshape(16, 1, R // 16))


def _merge_body(x_ref, nv_ref, ids_ref, slots_ref, out_ref,
                rowbuf0, rowbuf1, idx0, idx1, ss0, ss1, tmpidx, tmpslot,
                fixslot, fixoidx, gsem0, gsem1, wsem0, wsem1):
  gsems = (gsem0, gsem1)
  wsems = (wsem0, wsem1)
  c = lax.axis_index("c")
  s = lax.axis_index("s")
  wid = s * NC + c
  jbase = wid * J_PER_W
  rowbufs = (rowbuf0, rowbuf1)
  idxs = (idx0, idx1)
  sss = (ss0, ss1)
  nchunks = J_PER_W // CH

  pltpu.sync_copy(ids_ref.at[pl.ds(jbase, CH)], idx0)
  pltpu.sync_copy(slots_ref.at[pl.ds(jbase, CH)], ss0)
  pltpu.async_copy(x_ref.at[idx0], rowbuf0, gsem0)

  def chunk_step(q2, cnt, b):
    q = 2 * q2 + b
    base = jbase + q * CH
    buf, obuf = rowbufs[b], rowbufs[1 - b]
    has_next = (q2 < nchunks // 2 - 1) if b == 1 else None

    def load_next():
      nbase = base + CH
      pltpu.sync_copy(ids_ref.at[pl.ds(nbase, CH)], idxs[1 - b])
      pltpu.sync_copy(slots_ref.at[pl.ds(nbase, CH)], sss[1 - b])

    def wait_prev_write():
      pltpu.make_async_copy(obuf, out_ref.at[pl.ds(0, CH)], wsems[1 - b]).wait()

    def start_next_gather():
      pltpu.async_copy(x_ref.at[idxs[1 - b]], obuf, gsems[1 - b])

    if b == 0:
      load_next()
      pl.when(q2 > 0)(wait_prev_write)
      start_next_gather()
    else:
      pl.when(q2 < nchunks // 2 - 1)(load_next)
      wait_prev_write()
      pl.when(q2 < nchunks // 2 - 1)(start_next_gather)

    def mkidx(k, cnt):
      pv = sss[b][pl.ds(k * L, L)]
      m = pv != DEFAULT_SLOT
      jt = base + k * L + _iota16()
      plsc.store_compressed(fixslot.at[pl.ds(cnt, L)], pv, mask=m)
      plsc.store_compressed(fixoidx.at[pl.ds(cnt, L)], jt, mask=m)
      return cnt + jnp.sum(m.astype(jnp.int32))
    cnt = lax.fori_loop(0, CH // L, mkidx, cnt)

    pltpu.make_async_copy(x_ref.at[idxs[b]], buf, gsems[b]).wait()
    pltpu.async_copy(buf, out_ref.at[pl.ds(base, CH)], wsems[b])
    return cnt

  def outer(q2, cnt):
    cnt = chunk_step(q2, cnt, 0)
    cnt = chunk_step(q2, cnt, 1)
    return cnt
  cnt = lax.fori_loop(0, nchunks // 2, outer, 0)
  pltpu.make_async_copy(rowbuf1, out_ref.at[pl.ds(0, CH)], wsem1).wait()

  @pl.when(cnt > 0)
  def _():
    z = jnp.zeros((L,), jnp.int32)
    b_slot = plsc.load_gather(fixslot, [z])
    b_oidx = plsc.load_gather(fixoidx, [z])

    def pf(k, _):
      fixslot[pl.ds(cnt + k * L, L)] = b_slot
      fixoidx[pl.ds(cnt + k * L, L)] = b_oidx
      return 0
    lax.fori_loop(0, CH // L, pf, 0)

  trips = lax.div(cnt + (CH - 1), CH)

  def fixb(q, _):
    def cp(k, _):
      tmpslot[pl.ds(k * L, L)] = fixslot[pl.ds(q * CH + k * L, L)]
      tmpidx[pl.ds(k * L, L)] = fixoidx[pl.ds(q * CH + k * L, L)]
      return 0
    lax.fori_loop(0, CH // L, cp, 0)
    pltpu.async_copy(nv_ref.at[tmpslot], rowbuf0, gsem0).wait()
    pltpu.async_copy(rowbuf0, out_ref.at[tmpidx], gsem0).wait()
    return 0
  lax.fori_loop(0, trips, fixb, 0)


def _sc_merge(x_flat, newvals, ids_to_save, slot_s):
  mesh = plsc.VectorSubcoreMesh(core_axis_name="c", subcore_axis_name="s")
  return pl.kernel(
      _merge_body,
      out_type=jax.ShapeDtypeStruct((J, DM), jnp.float32),
      mesh=mesh,
      compiler_params=pltpu.CompilerParams(needs_layout_passes=False),
      scratch_types=[
          pltpu.VMEM((CH, DM), jnp.float32),
          pltpu.VMEM((CH, DM), jnp.float32),
          pltpu.VMEM((CH,), jnp.int32),
          pltpu.VMEM((CH,), jnp.int32),
          pltpu.VMEM((CH,), jnp.int32),
          pltpu.VMEM((CH,), jnp.int32),
          pltpu.VMEM((CH,), jnp.int32),
          pltpu.VMEM((CH,), jnp.int32),
          pltpu.VMEM((FIX_CAP,), jnp.int32),
          pltpu.VMEM((FIX_CAP,), jnp.int32),
          pltpu.SemaphoreType.DMA,
          pltpu.SemaphoreType.DMA,
          pltpu.SemaphoreType.DMA,
          pltpu.SemaphoreType.DMA,
      ],
  )(x_flat, newvals, ids_to_save, slot_s)


@jax.jit
def kernel(x, ids_to_save, ids_to_reduce, W):
  B, S, dm = x.shape
  x_flat = x.reshape(-1, dm)
  reduced, xt1, cnt_r, slot_s = _sc_gather_pos(
      x_flat, ids_to_reduce, ids_to_save)
  newvals = _tc_matmul(reduced, W, xt1, cnt_r)
  out = _sc_merge(x_flat, newvals, ids_to_save, slot_s)
  return out.reshape(B, -1, dm)


# speedup vs baseline: 3.2465x; 1.0012x over previous
import jax
import jax.numpy as jnp
from jax import lax
from jax.experimental import pallas as pl
from jax.experimental.pallas import tpu as pltpu
from jax.experimental.pallas import tpu_sc as plsc

NC = 2
NS = 16
L = 16

N = 32768
DM = 768
HALF = DM // 2
R = 4096
J = 28672

DEFAULT_SLOT = R
TOK_PER_SUB = N // NS
R_PER_TILE = R // (NC * NS)
J_PER_TILE = J // (NC * NS)
CHA = 64
J_PER_W = J // (NC * NS)
CH = 64
FIX_CAP = J_PER_W + CH


def _iota16():
  return lax.iota(jnp.int32, L)


def _gather_pos_body(x_ref, idr_ref, ids_ref,
                     red_ref, xt1_ref, cntr_ref, slots_ref,
                     tbuf, posslice, cntslice, pos_local, cnt_local,
                     idxbuf, rowbuf, lkpbuf, cntf,
                     shared_pos, shared_cnt, sem):
  c = lax.axis_index("c")
  s = lax.axis_index("s")
  wid = s * NC + c

  def gchunk(q, _):
    base = wid * R_PER_TILE + q * CHA
    pltpu.sync_copy(idr_ref.at[pl.ds(base, CHA)], idxbuf)
    pltpu.async_copy(x_ref.at[idxbuf], rowbuf, sem).wait()
    pltpu.sync_copy(rowbuf, red_ref.at[pl.ds(base, CHA)])

    def bump(k, _):
      idxbuf[pl.ds(k * L, L)] = idxbuf[pl.ds(k * L, L)] + 1
      return 0
    lax.fori_loop(0, CHA // L, bump, 0)
    pltpu.async_copy(x_ref.at[idxbuf], rowbuf, sem).wait()
    pltpu.sync_copy(rowbuf, xt1_ref.at[pl.ds(base, CHA)])
    return 0
  lax.fori_loop(0, R_PER_TILE // CHA, gchunk, 0)

  pltpu.sync_copy(idr_ref, tbuf)
  lo = s * TOK_PER_SUB

  def init_body(k, _):
    posslice[pl.ds(k * L, L)] = jnp.full((L,), DEFAULT_SLOT, jnp.int32)
    cntslice[pl.ds(k * L, L)] = jnp.zeros((L,), jnp.int32)
    return 0
  lax.fori_loop(0, TOK_PER_SUB // L, init_body, 0)

  def scat_body(k, _):
    tv = tbuf[pl.ds(k * L, L)] + 1
    sl = _iota16() + k * L
    m = (tv >= lo) & (tv < lo + TOK_PER_SUB)
    idx = jnp.where(m, tv - lo, 0)
    plsc.store_scatter(posslice, [idx], sl, mask=m)
    plsc.addupdate_scatter(cntslice, [idx], jnp.ones((L,), jnp.int32), mask=m)
    return 0
  lax.fori_loop(0, R // L, scat_body, 0)

  pltpu.sync_copy(posslice, shared_pos.at[pl.ds(lo, TOK_PER_SUB)])
  pltpu.sync_copy(cntslice, shared_cnt.at[pl.ds(lo, TOK_PER_SUB)])
  plsc.subcore_barrier()
  pltpu.sync_copy(shared_pos, pos_local)
  pltpu.sync_copy(shared_cnt, cnt_local)

  def lkr(k, _):
    tv = tbuf[pl.ds(wid * R_PER_TILE + k * L, L)] + 1
    cv = plsc.load_gather(cnt_local, [tv])
    cntf[pl.ds(k * L, L)] = cv.astype(jnp.float32)
    return 0
  lax.fori_loop(0, R_PER_TILE // L, lkr, 0)
  pltpu.sync_copy(cntf, cntr_ref.at[pl.ds(wid * R_PER_TILE, R_PER_TILE)])

  pltpu.sync_copy(ids_ref.at[pl.ds(wid * J_PER_TILE, J_PER_TILE)], lkpbuf)

  def lks(k, _):
    sv = lkpbuf[pl.ds(k * L, L)]
    lkpbuf[pl.ds(k * L, L)] = plsc.load_gather(pos_local, [sv])
    return 0
  lax.fori_loop(0, J_PER_TILE // L, lks, 0)
  pltpu.sync_copy(lkpbuf, slots_ref.at[pl.ds(wid * J_PER_TILE, J_PER_TILE)])


def _sc_gather_pos(x_flat, ids_to_reduce, ids_to_save):
  mesh = plsc.VectorSubcoreMesh(core_axis_name="c", subcore_axis_name="s")
  return pl.kernel(
      _gather_pos_body,
      out_type=[
          jax.ShapeDtypeStruct((R, DM), jnp.float32),
          jax.ShapeDtypeStruct((R, DM), jnp.float32),
          jax.ShapeDtypeStruct((R,), jnp.float32),
          jax.ShapeDtypeStruct((J,), jnp.int32),
      ],
      mesh=mesh,
      compiler_params=pltpu.CompilerParams(needs_layout_passes=False),
      scratch_types=[
          pltpu.VMEM((R,), jnp.int32),
          pltpu.VMEM((TOK_PER_SUB,), jnp.int32),
          pltpu.VMEM((TOK_PER_SUB,), jnp.int32),
          pltpu.VMEM((N,), jnp.int32),
          pltpu.VMEM((N,), jnp.int32),
          pltpu.VMEM((CHA,), jnp.int32),
          pltpu.VMEM((CHA, DM), jnp.float32),
          pltpu.VMEM((J_PER_TILE,), jnp.int32),
          pltpu.VMEM((R_PER_TILE,), jnp.float32),
          pltpu.VMEM_SHARED((N,), jnp.int32),
          pltpu.VMEM_SHARED((N,), jnp.int32),
          pltpu.SemaphoreType.DMA,
      ],
  )(x_flat, ids_to_reduce, ids_to_save)


def _mm_body(a_ref, w_ref, xt1_ref, cnt_ref, o_ref):
  prod = lax.dot_general(
      a_ref[...], w_ref[...], (((1,), (1,)), ((), ())),
      preferred_element_type=jnp.float32)
  o_ref[...] = prod * cnt_ref[0, 0, :][:, None] + xt1_ref[...]


def _tc_matmul(reduced, w, xt1, cnt_r):
  return pl.pallas_call(
      _mm_body,
      grid=(16,),
      in_specs=[
          pl.BlockSpec((R // 16, DM), lambda i: (i, 0)),
          pl.BlockSpec((DM, DM), lambda i: (0, 0)),
          pl.BlockSpec((R // 16, DM), lambda i: (i, 0)),
          pl.BlockSpec((1, 1, R // 16), lambda i: (i, 0, 0)),
      ],
      out_specs=pl.BlockSpec((R // 16, DM), lambda i: (i, 0)),
      out_shape=jax.ShapeDtypeStruct((R, DM), jnp.float32),
  )(reduced, w, xt1, cnt_r.reshape(16, 1, R // 16))


def _bulk_body(x_ref, ids_ref, out_ref,
               rowbuf0, rowbuf1, idx0, idx1,
               gsem0, gsem1, wsem0, wsem1):
  c = lax.axis_index("c")
  s = lax.axis_index("s")
  wid = s * NC + c
  jbase = wid * J_PER_W
  rowbufs = (rowbuf0, rowbuf1)
  idxs = (idx0, idx1)
  gsems = (gsem0, gsem1)
  wsems = (wsem0, wsem1)
  nchunks = J_PER_W // CH

  pltpu.sync_copy(ids_ref.at[pl.ds(jbase, CH)], idx0)
  pltpu.async_copy(x_ref.at[idx0], rowbuf0, gsem0)

  def chunk_step(q2, b):
    q = 2 * q2 + b
    base = jbase + q * CH
    buf, obuf = rowbufs[b], rowbufs[1 - b]

    def load_next():
      pltpu.sync_copy(ids_ref.at[pl.ds(base + CH, CH)], idxs[1 - b])

    def wait_prev_write():
      pltpu.make_async_copy(obuf, out_ref.at[pl.ds(0, CH)], wsems[1 - b]).wait()

    def start_next_gather():
      pltpu.async_copy(x_ref.at[idxs[1 - b]], obuf, gsems[1 - b])

    if b == 0:
      load_next()
      pl.when(q2 > 0)(wait_prev_write)
      start_next_gather()
    else:
      pl.when(q2 < nchunks // 2 - 1)(load_next)
      wait_prev_write()
      pl.when(q2 < nchunks // 2 - 1)(start_next_gather)

    pltpu.make_async_copy(x_ref.at[idxs[b]], buf, gsems[b]).wait()
    pltpu.async_copy(buf, out_ref.at[pl.ds(base, CH)], wsems[b])

  def outer(q2, carry):
    chunk_step(q2, 0)
    chunk_step(q2, 1)
    return carry
  lax.fori_loop(0, nchunks // 2, outer, 0)
  pltpu.make_async_copy(rowbuf1, out_ref.at[pl.ds(0, CH)], wsem1).wait()


def _sc_bulk(x_flat, ids_to_save):
  mesh = plsc.VectorSubcoreMesh(core_axis_name="c", subcore_axis_name="s")
  return pl.kernel(
      _bulk_body,
      out_type=jax.ShapeDtypeStruct((J, DM), jnp.float32),
      mesh=mesh,
      compiler_params=pltpu.CompilerParams(needs_layout_passes=False),
      scratch_types=[
          pltpu.VMEM((CH, DM), jnp.float32),
          pltpu.VMEM((CH, DM), jnp.float32),
          pltpu.VMEM((CH,), jnp.int32),
          pltpu.VMEM((CH,), jnp.int32),
          pltpu.SemaphoreType.DMA,
          pltpu.SemaphoreType.DMA,
          pltpu.SemaphoreType.DMA,
          pltpu.SemaphoreType.DMA,
      ],
  )(x_flat, ids_to_save)


def _fixup_body(nv_ref, slots_ref, out_ref,
                rowbuf, sschunk, tmpidx, tmpslot, fixslot, fixoidx, sem):
  c = lax.axis_index("c")
  s = lax.axis_index("s")
  wid = s * NC + c
  jbase = wid * J_PER_W

  def scan_chunk(q, cnt):
    base = jbase + q * CH
    pltpu.sync_copy(slots_ref.at[pl.ds(base, CH)], sschunk)

    def mkidx(k, cnt):
      pv = sschunk[pl.ds(k * L, L)]
      m = pv != DEFAULT_SLOT
      jt = base + k * L + _iota16()
      plsc.store_compressed(fixslot.at[pl.ds(cnt, L)], pv, mask=m)
      plsc.store_compressed(fixoidx.at[pl.ds(cnt, L)], jt, mask=m)
      return cnt + jnp.sum(m.astype(jnp.int32))
    return lax.fori_loop(0, CH // L, mkidx, cnt)
  cnt = lax.fori_loop(0, J_PER_W // CH, scan_chunk, 0)

  @pl.when(cnt > 0)
  def _():
    z = jnp.zeros((L,), jnp.int32)
    b_slot = plsc.load_gather(fixslot, [z])
    b_oidx = plsc.load_gather(fixoidx, [z])

    def pf(k, _):
      fixslot[pl.ds(cnt + k * L, L)] = b_slot
      fixoidx[pl.ds(cnt + k * L, L)] = b_oidx
      return 0
    lax.fori_loop(0, CH // L, pf, 0)

  trips = lax.div(cnt + (CH - 1), CH)

  def fixb(q, _):
    def cp(k, _):
      tmpslot[pl.ds(k * L, L)] = fixslot[pl.ds(q * CH + k * L, L)]
      tmpidx[pl.ds(k * L, L)] = fixoidx[pl.ds(q * CH + k * L, L)]
      return 0
    lax.fori_loop(0, CH // L, cp, 0)
    pltpu.async_copy(nv_ref.at[tmpslot], rowbuf, sem).wait()
    pltpu.async_copy(rowbuf, out_ref.at[tmpidx], sem).wait()
    return 0
  lax.fori_loop(0, trips, fixb, 0)


def _sc_fixup(newvals, slot_s, out_ref):
  mesh = plsc.VectorSubcoreMesh(core_axis_name="c", subcore_axis_name="s")
  return pl.kernel(
      _fixup_body,
      out_type=(),
      mesh=mesh,
      compiler_params=pltpu.CompilerParams(needs_layout_passes=False),
      scratch_types=[
          pltpu.VMEM((CH, DM), jnp.float32),
          pltpu.VMEM((CH,), jnp.int32),
          pltpu.VMEM((CH,), jnp.int32),
          pltpu.VMEM((CH,), jnp.int32),
          pltpu.VMEM((FIX_CAP,), jnp.int32),
          pltpu.VMEM((FIX_CAP,), jnp.int32),
          pltpu.SemaphoreType.DMA,
      ],
  )(newvals, slot_s, out_ref)


@jax.jit
def kernel(x, ids_to_save, ids_to_reduce, W):
  B, S, dm = x.shape
  x_flat = x.reshape(-1, dm)
  reduced, xt1, cnt_r, slot_s = _sc_gather_pos(
      x_flat, ids_to_reduce, ids_to_save)
  newvals = _tc_matmul(reduced, W, xt1, cnt_r)
  bulk = _sc_bulk(x_flat, ids_to_save)
  out_ref = jax.new_ref(bulk)
  _sc_fixup(newvals, slot_s, out_ref)
  return out_ref[...].reshape(B, -1, dm)


# speedup vs baseline: 3.4804x; 1.0721x over previous
import jax
import jax.numpy as jnp
from jax import lax
from jax.experimental import pallas as pl
from jax.experimental.pallas import tpu as pltpu
from jax.experimental.pallas import tpu_sc as plsc

NC = 2
NS = 16
L = 16

N = 32768
DM = 768
R = 4096
J = 28672

DEFAULT_SLOT = R
TOK_PER_SUB = N // NS
R_PER_TILE = R // (NC * NS)
J_PER_TILE = J // (NC * NS)
CHA = 32
J_PER_W = J // (NC * NS)
CH = 64
FIX_CAP = J_PER_W + CH


def _iota16():
  return lax.iota(jnp.int32, L)


def _gather_pos_body(x_ref, idr_ref, ids_ref,
                     red_ref, xt1_ref, cntr_ref, slots_ref,
                     tbuf, posslice, cntslice, pos_local, cnt_local,
                     idx0, idx1, rowbuf0, rowbuf1, lkpbuf, cntf,
                     shared_pos, shared_cnt,
                     gsem0, gsem1, wsem0, wsem1):
  c = lax.axis_index("c")
  s = lax.axis_index("s")
  wid = s * NC + c
  rowbufs = (rowbuf0, rowbuf1)
  idxs = (idx0, idx1)
  gsems = (gsem0, gsem1)
  wsems = (wsem0, wsem1)

  pltpu.sync_copy(idr_ref, tbuf)
  base0 = wid * R_PER_TILE

  def load_idx(q, b):
    off = 1 if q >= 4 else 0
    for k in range(CHA // L):
      pos = base0 + (q % 4) * CHA + k * L
      idxs[b][pl.ds(k * L, L)] = tbuf[pl.ds(pos, L)] + off

  def dst_slice(q):
    ref = red_ref if q < 4 else xt1_ref
    return ref.at[pl.ds(base0 + (q % 4) * CHA, CHA)]

  def build_pos_cnt():
    lo = s * TOK_PER_SUB

    def init_body(k, _):
      posslice[pl.ds(k * L, L)] = jnp.full((L,), DEFAULT_SLOT, jnp.int32)
      cntslice[pl.ds(k * L, L)] = jnp.zeros((L,), jnp.int32)
      return 0
    lax.fori_loop(0, TOK_PER_SUB // L, init_body, 0)

    def scat_body(k, _):
      tv = tbuf[pl.ds(k * L, L)] + 1
      sl = _iota16() + k * L
      m = (tv >= lo) & (tv < lo + TOK_PER_SUB)
      idx = jnp.where(m, tv - lo, 0)
      plsc.store_scatter(posslice, [idx], sl, mask=m)
      plsc.addupdate_scatter(cntslice, [idx], jnp.ones((L,), jnp.int32),
                             mask=m)
      return 0
    lax.fori_loop(0, R // L, scat_body, 0)

  load_idx(0, 0)
  pltpu.async_copy(x_ref.at[idx0], rowbuf0, gsem0)
  for q in range(8):
    b = q & 1
    if q < 7:
      load_idx(q + 1, 1 - b)
      if q >= 1:
        pltpu.make_async_copy(rowbufs[1 - b], dst_slice(q - 1),
                              wsems[1 - b]).wait()
      pltpu.async_copy(x_ref.at[idxs[1 - b]], rowbufs[1 - b], gsems[1 - b])
    if q == 0:
      build_pos_cnt()
    pltpu.make_async_copy(x_ref.at[idxs[b]], rowbufs[b], gsems[b]).wait()
    pltpu.async_copy(rowbufs[b], dst_slice(q), wsems[b])
  pltpu.make_async_copy(rowbuf0, dst_slice(6), wsem0).wait()
  pltpu.make_async_copy(rowbuf1, dst_slice(7), wsem1).wait()

  lo = s * TOK_PER_SUB
  pltpu.sync_copy(posslice, shared_pos.at[pl.ds(lo, TOK_PER_SUB)])
  pltpu.sync_copy(cntslice, shared_cnt.at[pl.ds(lo, TOK_PER_SUB)])
  plsc.subcore_barrier()
  pltpu.sync_copy(shared_pos, pos_local)
  pltpu.sync_copy(shared_cnt, cnt_local)

  def lkr(k, _):
    tv = tbuf[pl.ds(base0 + k * L, L)] + 1
    cv = plsc.load_gather(cnt_local, [tv])
    cntf[pl.ds(k * L, L)] = cv.astype(jnp.float32)
    return 0
  lax.fori_loop(0, R_PER_TILE // L, lkr, 0)
  pltpu.sync_copy(cntf, cntr_ref.at[pl.ds(base0, R_PER_TILE)])

  pltpu.sync_copy(ids_ref.at[pl.ds(wid * J_PER_TILE, J_PER_TILE)], lkpbuf)

  def lks(k, _):
    sv = lkpbuf[pl.ds(k * L, L)]
    lkpbuf[pl.ds(k * L, L)] = plsc.load_gather(pos_local, [sv])
    return 0
  lax.fori_loop(0, J_PER_TILE // L, lks, 0)
  pltpu.sync_copy(lkpbuf, slots_ref.at[pl.ds(wid * J_PER_TILE, J_PER_TILE)])


def _sc_gather_pos(x_flat, ids_to_reduce, ids_to_save):
  mesh = plsc.VectorSubcoreMesh(core_axis_name="c", subcore_axis_name="s")
  return pl.kernel(
      _gather_pos_body,
      out_type=[
          jax.ShapeDtypeStruct((R, DM), jnp.float32),
          jax.ShapeDtypeStruct((R, DM), jnp.float32),
          jax.ShapeDtypeStruct((R,), jnp.float32),
          jax.ShapeDtypeStruct((J,), jnp.int32),
      ],
      mesh=mesh,
      compiler_params=pltpu.CompilerParams(needs_layout_passes=False),
      scratch_types=[
          pltpu.VMEM((R,), jnp.int32),
          pltpu.VMEM((TOK_PER_SUB,), jnp.int32),
          pltpu.VMEM((TOK_PER_SUB,), jnp.int32),
          pltpu.VMEM((N,), jnp.int32),
          pltpu.VMEM((N,), jnp.int32),
          pltpu.VMEM((CHA,), jnp.int32),
          pltpu.VMEM((CHA,), jnp.int32),
          pltpu.VMEM((CHA, DM), jnp.float32),
          pltpu.VMEM((CHA, DM), jnp.float32),
          pltpu.VMEM((J_PER_TILE,), jnp.int32),
          pltpu.VMEM((R_PER_TILE,), jnp.float32),
          pltpu.VMEM_SHARED((N,), jnp.int32),
          pltpu.VMEM_SHARED((N,), jnp.int32),
          pltpu.SemaphoreType.DMA,
          pltpu.SemaphoreType.DMA,
          pltpu.SemaphoreType.DMA,
          pltpu.SemaphoreType.DMA,
      ],
  )(x_flat, ids_to_reduce, ids_to_save)


def _mm_body(a_ref, w_ref, xt1_ref, cnt_ref, o_ref):
  prod = lax.dot_general(
      a_ref[...], w_ref[...], (((1,), (1,)), ((), ())),
      preferred_element_type=jnp.float32)
  o_ref[...] = prod * cnt_ref[0, 0, :][:, None] + xt1_ref[...]


def _tc_matmul(reduced, w, xt1, cnt_r):
  return pl.pallas_call(
      _mm_body,
      grid=(16,),
      in_specs=[
          pl.BlockSpec((R // 16, DM), lambda i: (i, 0)),
          pl.BlockSpec((DM, DM), lambda i: (0, 0)),
          pl.BlockSpec((R // 16, DM), lambda i: (i, 0)),
          pl.BlockSpec((1, 1, R // 16), lambda i: (i, 0, 0)),
      ],
      out_specs=pl.BlockSpec((R // 16, DM), lambda i: (i, 0)),
      out_shape=jax.ShapeDtypeStruct((R, DM), jnp.float32),
  )(reduced, w, xt1, cnt_r.reshape(16, 1, R // 16))


def _bulk_body(x_ref, ids_ref, out_ref,
               rowbuf0, rowbuf1, idsbuf, idx0, idx1,
               gsem0, gsem1, wsem0, wsem1):
  c = lax.axis_index("c")
  s = lax.axis_index("s")
  wid = s * NC + c
  jbase = wid * J_PER_W
  rowbufs = (rowbuf0, rowbuf1)
  idxs = (idx0, idx1)
  gsems = (gsem0, gsem1)
  wsems = (wsem0, wsem1)
  nchunks = J_PER_W // CH

  pltpu.sync_copy(ids_ref.at[pl.ds(jbase, J_PER_W)], idsbuf)

  def load_idx(q, b):
    def cp(k, _):
      idxs[b][pl.ds(k * L, L)] = idsbuf[pl.ds(q * CH + k * L, L)]
      return 0
    lax.fori_loop(0, CH // L, cp, 0)

  load_idx(0, 0)
  pltpu.async_copy(x_ref.at[idx0], rowbuf0, gsem0)

  def chunk_step(q2, b):
    q = 2 * q2 + b
    base = jbase + q * CH
    buf, obuf = rowbufs[b], rowbufs[1 - b]

    def load_next():
      load_idx(q + 1, 1 - b)

    def wait_prev_write():
      pltpu.make_async_copy(obuf, out_ref.at[pl.ds(0, CH)], wsems[1 - b]).wait()

    def start_next_gather():
      pltpu.async_copy(x_ref.at[idxs[1 - b]], obuf, gsems[1 - b])

    if b == 0:
      load_next()
      pl.when(q2 > 0)(wait_prev_write)
      start_next_gather()
    else:
      pl.when(q2 < nchunks // 2 - 1)(load_next)
      wait_prev_write()
      pl.when(q2 < nchunks // 2 - 1)(start_next_gather)

    pltpu.make_async_copy(x_ref.at[idxs[b]], buf, gsems[b]).wait()
    pltpu.async_copy(buf, out_ref.at[pl.ds(base, CH)], wsems[b])

  def outer(q2, carry):
    chunk_step(q2, 0)
    chunk_step(q2, 1)
    return carry
  lax.fori_loop(0, nchunks // 2, outer, 0)
  pltpu.make_async_copy(rowbuf1, out_ref.at[pl.ds(0, CH)], wsem1).wait()


def _sc_bulk(x_flat, ids_to_save):
  mesh = plsc.VectorSubcoreMesh(core_axis_name="c", subcore_axis_name="s")
  return pl.kernel(
      _bulk_body,
      out_type=jax.ShapeDtypeStruct((J, DM), jnp.float32),
      mesh=mesh,
      compiler_params=pltpu.CompilerParams(needs_layout_passes=False),
      scratch_types=[
          pltpu.VMEM((CH, DM), jnp.float32),
          pltpu.VMEM((CH, DM), jnp.float32),
          pltpu.VMEM((J_PER_W,), jnp.int32),
          pltpu.VMEM((CH,), jnp.int32),
          pltpu.VMEM((CH,), jnp.int32),
          pltpu.SemaphoreType.DMA,
          pltpu.SemaphoreType.DMA,
          pltpu.SemaphoreType.DMA,
          pltpu.SemaphoreType.DMA,
      ],
  )(x_flat, ids_to_save)


def _fixup_body(nv_ref, slots_ref, out_ref,
                rowbuf0, rowbuf1, ssbuf,
                tidx0, tidx1, tslot0, tslot1, fixslot, fixoidx,
                gsem0, gsem1, wsem0, wsem1):
  c = lax.axis_index("c")
  s = lax.axis_index("s")
  wid = s * NC + c
  jbase = wid * J_PER_W
  rowbufs = (rowbuf0, rowbuf1)
  tidxs = (tidx0, tidx1)
  tslots = (tslot0, tslot1)
  gsems = (gsem0, gsem1)
  wsems = (wsem0, wsem1)

  pltpu.sync_copy(slots_ref.at[pl.ds(jbase, J_PER_W)], ssbuf)

  def mkidx(k, cnt):
    pv = ssbuf[pl.ds(k * L, L)]
    m = pv != DEFAULT_SLOT
    jt = jbase + k * L + _iota16()
    plsc.store_compressed(fixslot.at[pl.ds(cnt, L)], pv, mask=m)
    plsc.store_compressed(fixoidx.at[pl.ds(cnt, L)], jt, mask=m)
    return cnt + jnp.sum(m.astype(jnp.int32))
  cnt = lax.fori_loop(0, J_PER_W // L, mkidx, 0)

  @pl.when(cnt > 0)
  def _():
    z = jnp.zeros((L,), jnp.int32)
    b_slot = plsc.load_gather(fixslot, [z])
    b_oidx = plsc.load_gather(fixoidx, [z])

    def pf(k, _):
      fixslot[pl.ds(cnt + k * L, L)] = b_slot
      fixoidx[pl.ds(cnt + k * L, L)] = b_oidx
      return 0
    lax.fori_loop(0, CH // L, pf, 0)

  trips = lax.div(cnt + (CH - 1), CH)

  def prep(q, b):
    def cp(k, _):
      tslots[b][pl.ds(k * L, L)] = fixslot[pl.ds(q * CH + k * L, L)]
      tidxs[b][pl.ds(k * L, L)] = fixoidx[pl.ds(q * CH + k * L, L)]
      return 0
    lax.fori_loop(0, CH // L, cp, 0)

  @pl.when(trips > 0)
  def _():
    prep(0, 0)
    pltpu.async_copy(nv_ref.at[tslot0], rowbuf0, gsem0)

  def fix_outer(q2, carry):
    for b in (0, 1):
      q = 2 * q2 + b

      @pl.when(q < trips)
      def _():
        @pl.when(q + 1 < trips)
        def _():
          prep(q + 1, 1 - b)

        @pl.when(q >= 1)
        def _():
          pltpu.make_async_copy(rowbufs[1 - b], out_ref.at[pl.ds(0, CH)],
                                wsems[1 - b]).wait()

        @pl.when(q + 1 < trips)
        def _():
          pltpu.async_copy(nv_ref.at[tslots[1 - b]], rowbufs[1 - b],
                           gsems[1 - b])

        pltpu.make_async_copy(nv_ref.at[tslots[b]], rowbufs[b],
                              gsems[b]).wait()
        pltpu.async_copy(rowbufs[b], out_ref.at[tidxs[b]], wsems[b])
    return carry
  lax.fori_loop(0, (J_PER_W // CH + 1) // 2, fix_outer, 0)

  @pl.when((trips > 0) & (lax.rem(trips - 1, 2) == 0))
  def _():
    pltpu.make_async_copy(rowbuf0, out_ref.at[pl.ds(0, CH)], wsem0).wait()

  @pl.when((trips > 0) & (lax.rem(trips - 1, 2) == 1))
  def _():
    pltpu.make_async_copy(rowbuf1, out_ref.at[pl.ds(0, CH)], wsem1).wait()


def _sc_fixup(newvals, slot_s, out_ref):
  mesh = plsc.VectorSubcoreMesh(core_axis_name="c", subcore_axis_name="s")
  return pl.kernel(
      _fixup_body,
      out_type=(),
      mesh=mesh,
      compiler_params=pltpu.CompilerParams(needs_layout_passes=False),
      scratch_types=[
          pltpu.VMEM((CH, DM), jnp.float32),
          pltpu.VMEM((CH, DM), jnp.float32),
          pltpu.VMEM((J_PER_W,), jnp.int32),
          pltpu.VMEM((CH,), jnp.int32),
          pltpu.VMEM((CH,), jnp.int32),
          pltpu.VMEM((CH,), jnp.int32),
          pltpu.VMEM((CH,), jnp.int32),
          pltpu.VMEM((FIX_CAP,), jnp.int32),
          pltpu.VMEM((FIX_CAP,), jnp.int32),
          pltpu.SemaphoreType.DMA,
          pltpu.SemaphoreType.DMA,
          pltpu.SemaphoreType.DMA,
          pltpu.SemaphoreType.DMA,
      ],
  )(newvals, slot_s, out_ref)


@jax.jit
def kernel(x, ids_to_save, ids_to_reduce, W):
  B, S, dm = x.shape
  x_flat = x.reshape(-1, dm)
  reduced, xt1, cnt_r, slot_s = _sc_gather_pos(
      x_flat, ids_to_reduce, ids_to_save)
  newvals = _tc_matmul(reduced, W, xt1, cnt_r)
  bulk = _sc_bulk(x_flat, ids_to_save)
  out_ref = jax.new_ref(bulk)
  _sc_fixup(newvals, slot_s, out_ref)
  return out_ref[...].reshape(B, -1, dm)


# speedup vs baseline: 3.5044x; 1.0069x over previous
import jax
import jax.numpy as jnp
from jax import lax
from jax.experimental import pallas as pl
from jax.experimental.pallas import tpu as pltpu
from jax.experimental.pallas import tpu_sc as plsc

NC = 2
NS = 16
L = 16

N = 32768
DM = 768
R = 4096
J = 28672

DEFAULT_SLOT = R
TOK_PER_SUB = N // NS
R_PER_TILE = R // (NC * NS)
J_PER_TILE = J // (NC * NS)
CHA = 32
J_PER_W = J // (NC * NS)
CH = 64
CHB = 32
FIX_CAP = J_PER_W + CH


def _iota16():
  return lax.iota(jnp.int32, L)


def _gather_pos_body(x_ref, idr_ref, ids_ref,
                     red_ref, xt1_ref, cntr_ref, slots_ref,
                     tbuf, posslice, cntslice, pos_local, cnt_local,
                     idx0, idx1, rowbuf0, rowbuf1, lkpbuf, cntf,
                     shared_pos, shared_cnt,
                     gsem0, gsem1, wsem0, wsem1):
  c = lax.axis_index("c")
  s = lax.axis_index("s")
  wid = s * NC + c
  rowbufs = (rowbuf0, rowbuf1)
  idxs = (idx0, idx1)
  gsems = (gsem0, gsem1)
  wsems = (wsem0, wsem1)

  pltpu.sync_copy(idr_ref, tbuf)
  base0 = wid * R_PER_TILE

  def load_idx(q, b):
    off = 1 if q >= 4 else 0
    for k in range(CHA // L):
      pos = base0 + (q % 4) * CHA + k * L
      idxs[b][pl.ds(k * L, L)] = tbuf[pl.ds(pos, L)] + off

  def dst_slice(q):
    ref = red_ref if q < 4 else xt1_ref
    return ref.at[pl.ds(base0 + (q % 4) * CHA, CHA)]

  def build_pos_cnt():
    lo = s * TOK_PER_SUB

    def init_body(k, _):
      posslice[pl.ds(k * L, L)] = jnp.full((L,), DEFAULT_SLOT, jnp.int32)
      cntslice[pl.ds(k * L, L)] = jnp.zeros((L,), jnp.int32)
      return 0
    lax.fori_loop(0, TOK_PER_SUB // L, init_body, 0)

    def scat_body(k, _):
      tv = tbuf[pl.ds(k * L, L)] + 1
      sl = _iota16() + k * L
      m = (tv >= lo) & (tv < lo + TOK_PER_SUB)
      idx = jnp.where(m, tv - lo, 0)
      plsc.store_scatter(posslice, [idx], sl, mask=m)
      plsc.addupdate_scatter(cntslice, [idx], jnp.ones((L,), jnp.int32),
                             mask=m)
      return 0
    lax.fori_loop(0, R // L, scat_body, 0)

  load_idx(0, 0)
  pltpu.async_copy(x_ref.at[idx0], rowbuf0, gsem0)
  for q in range(8):
    b = q & 1
    if q < 7:
      load_idx(q + 1, 1 - b)
      if q >= 1:
        pltpu.make_async_copy(rowbufs[1 - b], dst_slice(q - 1),
                              wsems[1 - b]).wait()
      pltpu.async_copy(x_ref.at[idxs[1 - b]], rowbufs[1 - b], gsems[1 - b])
    if q == 0:
      build_pos_cnt()
    pltpu.make_async_copy(x_ref.at[idxs[b]], rowbufs[b], gsems[b]).wait()
    pltpu.async_copy(rowbufs[b], dst_slice(q), wsems[b])
  pltpu.make_async_copy(rowbuf0, dst_slice(6), wsem0).wait()
  pltpu.make_async_copy(rowbuf1, dst_slice(7), wsem1).wait()

  lo = s * TOK_PER_SUB
  pltpu.sync_copy(posslice, shared_pos.at[pl.ds(lo, TOK_PER_SUB)])
  pltpu.sync_copy(cntslice, shared_cnt.at[pl.ds(lo, TOK_PER_SUB)])
  plsc.subcore_barrier()
  pltpu.sync_copy(shared_pos, pos_local)
  pltpu.sync_copy(shared_cnt, cnt_local)

  def lkr(k, _):
    tv = tbuf[pl.ds(base0 + k * L, L)] + 1
    cv = plsc.load_gather(cnt_local, [tv])
    cntf[pl.ds(k * L, L)] = cv.astype(jnp.float32)
    return 0
  lax.fori_loop(0, R_PER_TILE // L, lkr, 0)
  pltpu.sync_copy(cntf, cntr_ref.at[pl.ds(base0, R_PER_TILE)])

  pltpu.sync_copy(ids_ref.at[pl.ds(wid * J_PER_TILE, J_PER_TILE)], lkpbuf)

  def lks(k, _):
    sv = lkpbuf[pl.ds(k * L, L)]
    lkpbuf[pl.ds(k * L, L)] = plsc.load_gather(pos_local, [sv])
    return 0
  lax.fori_loop(0, J_PER_TILE // L, lks, 0)
  pltpu.sync_copy(lkpbuf, slots_ref.at[pl.ds(wid * J_PER_TILE, J_PER_TILE)])


def _sc_gather_pos(x_flat, ids_to_reduce, ids_to_save):
  mesh = plsc.VectorSubcoreMesh(core_axis_name="c", subcore_axis_name="s")
  return pl.kernel(
      _gather_pos_body,
      out_type=[
          jax.ShapeDtypeStruct((R, DM), jnp.float32),
          jax.ShapeDtypeStruct((R, DM), jnp.float32),
          jax.ShapeDtypeStruct((R,), jnp.float32),
          jax.ShapeDtypeStruct((J,), jnp.int32),
      ],
      mesh=mesh,
      compiler_params=pltpu.CompilerParams(needs_layout_passes=False),
      scratch_types=[
          pltpu.VMEM((R,), jnp.int32),
          pltpu.VMEM((TOK_PER_SUB,), jnp.int32),
          pltpu.VMEM((TOK_PER_SUB,), jnp.int32),
          pltpu.VMEM((N,), jnp.int32),
          pltpu.VMEM((N,), jnp.int32),
          pltpu.VMEM((CHA,), jnp.int32),
          pltpu.VMEM((CHA,), jnp.int32),
          pltpu.VMEM((CHA, DM), jnp.float32),
          pltpu.VMEM((CHA, DM), jnp.float32),
          pltpu.VMEM((J_PER_TILE,), jnp.int32),
          pltpu.VMEM((R_PER_TILE,), jnp.float32),
          pltpu.VMEM_SHARED((N,), jnp.int32),
          pltpu.VMEM_SHARED((N,), jnp.int32),
          pltpu.SemaphoreType.DMA,
          pltpu.SemaphoreType.DMA,
          pltpu.SemaphoreType.DMA,
          pltpu.SemaphoreType.DMA,
      ],
  )(x_flat, ids_to_reduce, ids_to_save)


def _mm_body(a_ref, w_ref, xt1_ref, cnt_ref, o_ref):
  prod = lax.dot_general(
      a_ref[...], w_ref[...], (((1,), (1,)), ((), ())),
      preferred_element_type=jnp.float32)
  o_ref[...] = prod * cnt_ref[0, 0, :][:, None] + xt1_ref[...]


def _tc_matmul(reduced, w, xt1, cnt_r):
  return pl.pallas_call(
      _mm_body,
      grid=(16,),
      in_specs=[
          pl.BlockSpec((R // 16, DM), lambda i: (i, 0)),
          pl.BlockSpec((DM, DM), lambda i: (0, 0)),
          pl.BlockSpec((R // 16, DM), lambda i: (i, 0)),
          pl.BlockSpec((1, 1, R // 16), lambda i: (i, 0, 0)),
      ],
      out_specs=pl.BlockSpec((R // 16, DM), lambda i: (i, 0)),
      out_shape=jax.ShapeDtypeStruct((R, DM), jnp.float32),
  )(reduced, w, xt1, cnt_r.reshape(16, 1, R // 16))


def _bulk_body(x_ref, ids_ref, out_ref,
               buf0, buf1, buf2, buf3, idsbuf, idx0, idx1, idx2, idx3,
               g0, g1, g2, g3, w0, w1, w2, w3):
  c = lax.axis_index("c")
  s = lax.axis_index("s")
  wid = s * NC + c
  jbase = wid * J_PER_W
  bufs = (buf0, buf1, buf2, buf3)
  idxs = (idx0, idx1, idx2, idx3)
  gsems = (g0, g1, g2, g3)
  wsems = (w0, w1, w2, w3)
  nchunks = J_PER_W // CHB

  pltpu.sync_copy(ids_ref.at[pl.ds(jbase, J_PER_W)], idsbuf)

  def load_idx(q, b):
    def cp(k, _):
      idxs[b][pl.ds(k * L, L)] = idsbuf[pl.ds(q * CHB + k * L, L)]
      return 0
    lax.fori_loop(0, CHB // L, cp, 0)

  for p in range(3):
    load_idx(p, p)
    pltpu.async_copy(x_ref.at[idxs[p]], bufs[p], gsems[p])

  def chunk_step(q2, qq):
    q = 4 * q2 + qq
    b = qq
    bn = (qq + 3) % 4
    base = jbase + q * CHB

    def feed():
      load_idx(q + 3, bn)

      def wait_prev_write():
        pltpu.make_async_copy(bufs[bn], out_ref.at[pl.ds(0, CHB)],
                              wsems[bn]).wait()
      if qq == 0:
        pl.when(q2 > 0)(wait_prev_write)
      else:
        wait_prev_write()
      pltpu.async_copy(x_ref.at[idxs[bn]], bufs[bn], gsems[bn])

    if qq == 0:
      feed()
    else:
      pl.when(q2 < 6)(feed)

    pltpu.make_async_copy(x_ref.at[idxs[b]], bufs[b], gsems[b]).wait()
    pltpu.async_copy(bufs[b], out_ref.at[pl.ds(base, CHB)], wsems[b])

  def outer(q2, carry):
    for qq in range(4):
      chunk_step(q2, qq)
    return carry
  lax.fori_loop(0, nchunks // 4, outer, 0)
  for p in range(4):
    q = 24 + p
    pltpu.make_async_copy(bufs[q % 4], out_ref.at[pl.ds(0, CHB)],
                          wsems[q % 4]).wait()


def _sc_bulk(x_flat, ids_to_save):
  mesh = plsc.VectorSubcoreMesh(core_axis_name="c", subcore_axis_name="s")
  return pl.kernel(
      _bulk_body,
      out_type=jax.ShapeDtypeStruct((J, DM), jnp.float32),
      mesh=mesh,
      compiler_params=pltpu.CompilerParams(needs_layout_passes=False),
      scratch_types=(
          [pltpu.VMEM((CHB, DM), jnp.float32)] * 4
          + [pltpu.VMEM((J_PER_W,), jnp.int32)]
          + [pltpu.VMEM((CHB,), jnp.int32)] * 4
          + [pltpu.SemaphoreType.DMA] * 8
      ),
  )(x_flat, ids_to_save)


def _fixup_body(nv_ref, slots_ref, out_ref,
                rowbuf0, rowbuf1, ssbuf,
                tidx0, tidx1, tslot0, tslot1, fixslot, fixoidx,
                gsem0, gsem1, wsem0, wsem1):
  c = lax.axis_index("c")
  s = lax.axis_index("s")
  wid = s * NC + c
  jbase = wid * J_PER_W
  rowbufs = (rowbuf0, rowbuf1)
  tidxs = (tidx0, tidx1)
  tslots = (tslot0, tslot1)
  gsems = (gsem0, gsem1)
  wsems = (wsem0, wsem1)

  pltpu.sync_copy(slots_ref.at[pl.ds(jbase, J_PER_W)], ssbuf)

  def mkidx(k, cnt):
    pv = ssbuf[pl.ds(k * L, L)]
    m = pv != DEFAULT_SLOT
    jt = jbase + k * L + _iota16()
    plsc.store_compressed(fixslot.at[pl.ds(cnt, L)], pv, mask=m)
    plsc.store_compressed(fixoidx.at[pl.ds(cnt, L)], jt, mask=m)
    return cnt + jnp.sum(m.astype(jnp.int32))
  cnt = lax.fori_loop(0, J_PER_W // L, mkidx, 0)

  @pl.when(cnt > 0)
  def _():
    z = jnp.zeros((L,), jnp.int32)
    b_slot = plsc.load_gather(fixslot, [z])
    b_oidx = plsc.load_gather(fixoidx, [z])

    def pf(k, _):
      fixslot[pl.ds(cnt + k * L, L)] = b_slot
      fixoidx[pl.ds(cnt + k * L, L)] = b_oidx
      return 0
    lax.fori_loop(0, CH // L, pf, 0)

  trips = lax.div(cnt + (CH - 1), CH)

  def prep(q, b):
    def cp(k, _):
      tslots[b][pl.ds(k * L, L)] = fixslot[pl.ds(q * CH + k * L, L)]
      tidxs[b][pl.ds(k * L, L)] = fixoidx[pl.ds(q * CH + k * L, L)]
      return 0
    lax.fori_loop(0, CH // L, cp, 0)

  @pl.when(trips > 0)
  def _():
    prep(0, 0)
    pltpu.async_copy(nv_ref.at[tslot0], rowbuf0, gsem0)

  def fix_outer(q2, carry):
    for b in (0, 1):
      q = 2 * q2 + b

      @pl.when(q < trips)
      def _():
        @pl.when(q + 1 < trips)
        def _():
          prep(q + 1, 1 - b)

        @pl.when(q >= 1)
        def _():
          pltpu.make_async_copy(rowbufs[1 - b], out_ref.at[pl.ds(0, CH)],
                                wsems[1 - b]).wait()

        @pl.when(q + 1 < trips)
        def _():
          pltpu.async_copy(nv_ref.at[tslots[1 - b]], rowbufs[1 - b],
                           gsems[1 - b])

        pltpu.make_async_copy(nv_ref.at[tslots[b]], rowbufs[b],
                              gsems[b]).wait()
        pltpu.async_copy(rowbufs[b], out_ref.at[tidxs[b]], wsems[b])
    return carry
  lax.fori_loop(0, (J_PER_W // CH + 1) // 2, fix_outer, 0)

  @pl.when((trips > 0) & (lax.rem(trips - 1, 2) == 0))
  def _():
    pltpu.make_async_copy(rowbuf0, out_ref.at[pl.ds(0, CH)], wsem0).wait()

  @pl.when((trips > 0) & (lax.rem(trips - 1, 2) == 1))
  def _():
    pltpu.make_async_copy(rowbuf1, out_ref.at[pl.ds(0, CH)], wsem1).wait()


def _sc_fixup(newvals, slot_s, out_ref):
  mesh = plsc.VectorSubcoreMesh(core_axis_name="c", subcore_axis_name="s")
  return pl.kernel(
      _fixup_body,
      out_type=(),
      mesh=mesh,
      compiler_params=pltpu.CompilerParams(needs_layout_passes=False),
      scratch_types=[
          pltpu.VMEM((CH, DM), jnp.float32),
          pltpu.VMEM((CH, DM), jnp.float32),
          pltpu.VMEM((J_PER_W,), jnp.int32),
          pltpu.VMEM((CH,), jnp.int32),
          pltpu.VMEM((CH,), jnp.int32),
          pltpu.VMEM((CH,), jnp.int32),
          pltpu.VMEM((CH,), jnp.int32),
          pltpu.VMEM((FIX_CAP,), jnp.int32),
          pltpu.VMEM((FIX_CAP,), jnp.int32),
          pltpu.SemaphoreType.DMA,
          pltpu.SemaphoreType.DMA,
          pltpu.SemaphoreType.DMA,
          pltpu.SemaphoreType.DMA,
      ],
  )(newvals, slot_s, out_ref)


@jax.jit
def kernel(x, ids_to_save, ids_to_reduce, W):
  B, S, dm = x.shape
  x_flat = x.reshape(-1, dm)
  reduced, xt1, cnt_r, slot_s = _sc_gather_pos(
      x_flat, ids_to_reduce, ids_to_save)
  newvals = _tc_matmul(reduced, W, xt1, cnt_r)
  bulk = _sc_bulk(x_flat, ids_to_save)
  out_ref = jax.new_ref(bulk)
  _sc_fixup(newvals, slot_s, out_ref)
  return out_ref[...].reshape(B, -1, dm)
